# Initial kernel scaffold; baseline (speedup 1.0000x reference)
import jax
import jax.numpy as jnp
from jax.experimental import pallas as pl


def kernel(z, W):
    raise NotImplementedError("write your pallas kernel here")


# speedup vs baseline: 1.3864x; 1.3864x over previous
import jax
import jax.numpy as jnp
from jax.experimental import pallas as pl
from jax.experimental.pallas import tpu as pltpu
from jax.experimental.pallas import tpu_sc as plsc

_N_E = 8192
_E_DIM = 256
_BETA = 0.25
_WD = 0.01
_TOK = 8192
_BM = 256


def _preprocess(z):
    sf = 2
    b, c, h, w = z.shape
    x = z[:, :, None, :, :]
    x = jax.image.resize(x, (b, c, 2, 2 * h, 2 * w), method='trilinear')
    bs, cc, d, hh, ww = x.shape
    x = x.reshape(bs, cc, d // sf, sf, hh // sf, sf, ww // sf, sf)
    x = jnp.transpose(x, (0, 1, 3, 5, 7, 2, 4, 6))
    x = x.reshape(bs, cc * sf ** 3, d // sf, hh // sf, ww // sf)
    x = x[:, :, 0, :, :]
    z2 = jnp.transpose(x, (0, 2, 3, 1))
    zf = z2.reshape(-1, _E_DIM)
    return zf


def _dist_argmin_body(zf_ref, zn_ref, w_ref, wn_ref, idx_ref):
    dot = jax.lax.dot_general(
        zf_ref[...].astype(jnp.bfloat16), w_ref[...].astype(jnp.bfloat16),
        dimension_numbers=(((1,), (1,)), ((), ())),
        preferred_element_type=jnp.float32)
    dist = (zn_ref[...] + wn_ref[...][None, :]) - 2.0 * dot
    m = jnp.min(dist, axis=1, keepdims=True)
    ks = jax.lax.broadcasted_iota(jnp.int32, dist.shape, 1)
    idx_ref[...] = jnp.min(jnp.where(dist == m, ks, jnp.int32(_N_E)), axis=1)


def _dist_argmin(zf, znorm, W, wnorm):
    grid = (_TOK // _BM,)
    return pl.pallas_call(
        _dist_argmin_body,
        grid=grid,
        in_specs=[
            pl.BlockSpec((_BM, _E_DIM), lambda i: (i, 0)),
            pl.BlockSpec((_BM, 1), lambda i: (i, 0)),
            pl.BlockSpec((_N_E, _E_DIM), lambda i: (0, 0)),
            pl.BlockSpec((_N_E,), lambda i: (0,)),
        ],
        out_specs=pl.BlockSpec((_BM,), lambda i: (i,)),
        out_shape=jax.ShapeDtypeStruct((_TOK,), jnp.int32),
    )(zf, znorm, W, wnorm)


def _sc_gather(W, idx):
    idx2 = idx.reshape(1, _TOK)
    mesh = plsc.VectorSubcoreMesh(core_axis_name="core",
                                  subcore_axis_name="subcore")

    @pl.kernel(out_type=jax.ShapeDtypeStruct((_TOK, _E_DIM), W.dtype),
               mesh=mesh)
    def k(w_hbm, i_hbm, o_hbm):
        def body(i_vmem, o_vmem):
            pltpu.sync_copy(w_hbm.at[i_vmem.at[0]], o_vmem)

        pltpu.emit_pipeline(
            body,
            grid=(_TOK // 128,),
            in_specs=[pl.BlockSpec((1, 128), index_map=lambda i: (0, i))],
            out_specs=[pl.BlockSpec((128, _E_DIM), index_map=lambda i: (i, 0))],
            core_axis_name=("core", "subcore"),
            dimension_semantics=(pltpu.PARALLEL,),
        )(i_hbm, o_hbm)

    return k(W, idx2)


def _loss_out_body(zf_ref, zq_ref, w_ref, loss_ref, avg_ref):
    zf = zf_ref[...]
    zq = zq_ref[...]
    w = w_ref[...]
    n = float(_TOK * _E_DIM)
    diff2 = (zq - zf) ** 2
    mse = jnp.sum(diff2) / n
    s_zq = jnp.sum(zq)
    s_z2 = jnp.sum(zf)
    s_zq2 = jnp.sum(zq * zq)
    s_z22 = jnp.sum(zf * zf)
    s_cross = jnp.sum(zq * zf)
    cov = s_cross - s_zq * s_z2 / n
    vzq = s_zq2 - s_zq * s_zq / n
    vz2 = s_z22 - s_z2 * s_z2 / n
    pear = 0.5 + 0.5 * (cov / (jnp.sqrt(vzq) * jnp.sqrt(vz2)))
    reg = _WD * jnp.max(jnp.sum(jnp.abs(w), axis=0))
    loss = (_BETA + 1.0) * mse + pear + reg
    loss_ref[...] = loss[None, None]
    ste = zf + (zq - zf)
    ci = jax.lax.broadcasted_iota(jnp.int32, (_E_DIM, _E_DIM // 8), 0)
    co = jax.lax.broadcasted_iota(jnp.int32, (_E_DIM, _E_DIM // 8), 1)
    pool = jnp.where(ci // 8 == co, 0.125, 0.0).astype(jnp.float32)
    avg_ref[...] = jax.lax.dot_general(
        ste, pool,
        dimension_numbers=(((1,), (0,)), ((), ())),
        preferred_element_type=jnp.float32,
        precision=jax.lax.Precision.HIGHEST)


def _loss_and_output(zf, zq, W):
    return pl.pallas_call(
        _loss_out_body,
        grid=(1,),
        in_specs=[
            pl.BlockSpec((_TOK, _E_DIM), lambda i: (0, 0)),
            pl.BlockSpec((_TOK, _E_DIM), lambda i: (0, 0)),
            pl.BlockSpec((_N_E, _E_DIM), lambda i: (0, 0)),
        ],
        out_specs=[
            pl.BlockSpec((1, 1), lambda i: (0, 0)),
            pl.BlockSpec((_TOK, _E_DIM // 8), lambda i: (0, 0)),
        ],
        out_shape=[
            jax.ShapeDtypeStruct((1, 1), jnp.float32),
            jax.ShapeDtypeStruct((_TOK, _E_DIM // 8), jnp.float32),
        ],
    )(zf, zq, W)


def kernel(z, W):
    zf = _preprocess(z)
    znorm = jnp.sum(zf ** 2, axis=1, keepdims=True)
    wnorm = jnp.sum(W ** 2, axis=1)
    idx = _dist_argmin(zf, znorm, W, wnorm)
    zq = _sc_gather(W, idx)
    loss, avg = _loss_and_output(zf, zq, W)
    b, c, h, w = z.shape
    z_q = avg.reshape(b, h, w, c).transpose(0, 3, 1, 2)
    return (z_q, loss.reshape(()), idx)
